# Initial kernel scaffold; baseline (speedup 1.0000x reference)
import jax
import jax.numpy as jnp
from jax.experimental import pallas as pl


def kernel(x, edge_index_0, edge_index_1, size_0, size_1, W1, b1, W2, b2, bn_gamma, bn_beta, prelu_a):
    raise NotImplementedError("write your pallas kernel here")


# speedup vs baseline: 16.4183x; 16.4183x over previous
import functools

import jax
import jax.numpy as jnp
from jax import lax
from jax.experimental import pallas as pl
from jax.experimental.pallas import tpu as pltpu
from jax.experimental.pallas import tpu_sc as plsc

_N = 10000
_E = 320000
_D = 128
_NC = 2
_NS = 16
_NW = _NC * _NS
_EW = _E // _NW
_CH = 128
_NFULL = _EW // _CH
_TAIL = _EW - _NFULL * _CH
_NP = 10240
_RPT = _NP // _NS
_ZCH = 128
_NZ = _RPT // _ZCH

_R = 1000
_G = _N // _R


def _sc_mesh():
    return plsc.VectorSubcoreMesh(
        core_axis_name="c", subcore_axis_name="s",
        num_cores=_NC, num_subcores=_NS)


def _sc_degrees(dst0, dst1, zrows, orows):
    @functools.partial(
        pl.kernel,
        out_type=(
            jax.ShapeDtypeStruct((_NC * _NP, _D), jnp.float32),
            jax.ShapeDtypeStruct((_NC * _NP, _D), jnp.float32),
        ),
        mesh=_sc_mesh(),
        scratch_types=[
            pltpu.VMEM((_CH,), jnp.int32),
            pltpu.VMEM((_TAIL,), jnp.int32),
            pltpu.VMEM((_CH, _D), jnp.float32),
            pltpu.VMEM((_TAIL, _D), jnp.float32),
            pltpu.VMEM((_ZCH, _D), jnp.float32),
            pltpu.VMEM_SHARED((_NP, _D), jnp.float32),
        ],
    )
    def k(dst0_h, dst1_h, z_h, o_h, out0_h, out1_h,
          idx_v, idxt_v, ones_v, onest_v, zbuf_v, acc_s):
        c = lax.axis_index("c")
        s = lax.axis_index("s")
        wid = c * _NS + s
        r0 = s * _RPT
        ebase = wid * _EW
        bt = ebase + _NFULL * _CH
        pltpu.sync_copy(z_h, zbuf_v)
        pltpu.sync_copy(o_h, ones_v)
        pltpu.sync_copy(o_h.at[pl.ds(0, _TAIL)], onest_v)

        for dst_h, out_h in ((dst0_h, out0_h), (dst1_h, out1_h)):
            for j in range(_NZ):
                pltpu.sync_copy(zbuf_v, acc_s.at[pl.ds(r0 + j * _ZCH, _ZCH)])
            plsc.subcore_barrier()

            @pl.loop(0, _NFULL)
            def body(j):
                b = ebase + j * _CH
                pltpu.sync_copy(dst_h.at[pl.ds(b, _CH)], idx_v)
                pltpu.sync_copy(ones_v, acc_s.at[idx_v], add=True)

            pltpu.sync_copy(dst_h.at[pl.ds(bt, _TAIL)], idxt_v)
            pltpu.sync_copy(onest_v, acc_s.at[idxt_v], add=True)

            plsc.subcore_barrier()
            for j in range(_NZ):
                rr = r0 + j * _ZCH
                pltpu.sync_copy(acc_s.at[pl.ds(rr, _ZCH)], zbuf_v)
                pltpu.sync_copy(zbuf_v, out_h.at[pl.ds(c * _NP + rr, _ZCH)])
            pltpu.sync_copy(z_h, zbuf_v)

    return k(dst0, dst1, zrows, orows)


def _sc_aggregate(src, dst, table, zrows):
    @functools.partial(
        pl.kernel,
        out_type=jax.ShapeDtypeStruct((_NC * _NP, _D), jnp.float32),
        mesh=_sc_mesh(),
        scratch_types=[
            pltpu.VMEM((_CH,), jnp.int32),
            pltpu.VMEM((_CH,), jnp.int32),
            pltpu.VMEM((_TAIL,), jnp.int32),
            pltpu.VMEM((_TAIL,), jnp.int32),
            pltpu.VMEM((_CH, _D), jnp.float32),
            pltpu.VMEM((_TAIL, _D), jnp.float32),
            pltpu.VMEM((_ZCH, _D), jnp.float32),
            pltpu.VMEM_SHARED((_NP, _D), jnp.float32),
            pltpu.SemaphoreType.DMA,
        ],
    )
    def k(src_h, dst_h, tab_h, z_h, out_h,
          si_v, di_v, sit_v, dit_v, rows_v, rowst_v, zbuf_v, acc_s, sem):
        c = lax.axis_index("c")
        s = lax.axis_index("s")
        wid = c * _NS + s
        r0 = s * _RPT
        pltpu.sync_copy(z_h, zbuf_v)
        for j in range(_NZ):
            pltpu.sync_copy(zbuf_v, acc_s.at[pl.ds(r0 + j * _ZCH, _ZCH)])
        plsc.subcore_barrier()

        ebase = wid * _EW

        @pl.loop(0, _NFULL)
        def body(j):
            b = ebase + j * _CH
            pltpu.sync_copy(src_h.at[pl.ds(b, _CH)], si_v)
            pltpu.sync_copy(dst_h.at[pl.ds(b, _CH)], di_v)
            pltpu.async_copy(tab_h.at[si_v], rows_v, sem).wait()
            pltpu.sync_copy(rows_v, acc_s.at[di_v], add=True)

        bt = ebase + _NFULL * _CH
        pltpu.sync_copy(src_h.at[pl.ds(bt, _TAIL)], sit_v)
        pltpu.sync_copy(dst_h.at[pl.ds(bt, _TAIL)], dit_v)
        pltpu.async_copy(tab_h.at[sit_v], rowst_v, sem).wait()
        pltpu.sync_copy(rowst_v, acc_s.at[dit_v], add=True)

        plsc.subcore_barrier()
        for j in range(_NZ):
            rr = r0 + j * _ZCH
            pltpu.sync_copy(acc_s.at[pl.ds(rr, _ZCH)], zbuf_v)
            pltpu.sync_copy(zbuf_v, out_h.at[pl.ds(c * _NP + rr, _ZCH)])

    return k(src, dst, table, zrows)


def _dinv(d0, d1):
    return lax.rsqrt(d0[:, 0:1] + d1[:, 0:1] + 1.0)


def _t1_scale(deg0a, deg0b, x):
    def body(d0_ref, d1_ref, x_ref, o_ref):
        o_ref[...] = x_ref[...] * _dinv(d0_ref[...], d1_ref[...])

    return pl.pallas_call(
        body,
        grid=(_G,),
        in_specs=[
            pl.BlockSpec((_R, _D), lambda i: (i, 0)),
            pl.BlockSpec((_R, _D), lambda i: (i, 0)),
            pl.BlockSpec((_R, _D), lambda i: (i, 0)),
        ],
        out_specs=pl.BlockSpec((_R, _D), lambda i: (i, 0)),
        out_shape=jax.ShapeDtypeStruct((_N, _D), jnp.float32),
    )(deg0a, deg0b, x)


def _t2_layer1(agg0a, agg0b, xp, deg0a, deg0b, W1, b1r):
    def body(a0_ref, a1_ref, xp_ref, d0_ref, d1_ref, w_ref, b_ref,
             y_ref, st_ref):
        dinv = _dinv(d0_ref[...], d1_ref[...])
        a = (a0_ref[...] + a1_ref[...] + xp_ref[...]) * dinv
        y = jnp.dot(a, w_ref[...], preferred_element_type=jnp.float32)
        y = y + b_ref[...]
        y_ref[...] = y

        @pl.when(pl.program_id(0) == 0)
        def _():
            st_ref[...] = jnp.zeros_like(st_ref)

        st_ref[0:1, :] += jnp.sum(y, axis=0, keepdims=True)
        st_ref[1:2, :] += jnp.sum(y * y, axis=0, keepdims=True)

    return pl.pallas_call(
        body,
        grid=(_G,),
        in_specs=[
            pl.BlockSpec((_R, _D), lambda i: (i, 0)),
            pl.BlockSpec((_R, _D), lambda i: (i, 0)),
            pl.BlockSpec((_R, _D), lambda i: (i, 0)),
            pl.BlockSpec((_R, _D), lambda i: (i, 0)),
            pl.BlockSpec((_R, _D), lambda i: (i, 0)),
            pl.BlockSpec((128, 256), lambda i: (0, 0)),
            pl.BlockSpec((1, 256), lambda i: (0, 0)),
        ],
        out_specs=[
            pl.BlockSpec((_R, 256), lambda i: (i, 0)),
            pl.BlockSpec((2, 256), lambda i: (0, 0)),
        ],
        out_shape=[
            jax.ShapeDtypeStruct((_N, 256), jnp.float32),
            jax.ShapeDtypeStruct((2, 256), jnp.float32),
        ],
    )(agg0a, agg0b, xp, deg0a, deg0b, W1, b1r)


def _t3_layer2_in(y, st, gammar, betar, par, W2, deg1a, deg1b):
    def body(y_ref, st_ref, g_ref, bt_ref, pa_ref, w_ref, d0_ref, d1_ref,
             o_ref):
        inv_n = 1.0 / _N
        mean = st_ref[0:1, :] * inv_n
        var = st_ref[1:2, :] * inv_n - mean * mean
        yb = (y_ref[...] - mean) * lax.rsqrt(var + 1e-5)
        yb = yb * g_ref[...] + bt_ref[...]
        a = pa_ref[0, 0]
        h = jnp.where(yb >= 0.0, yb, a * yb)
        z = jnp.dot(h, w_ref[...], preferred_element_type=jnp.float32)
        o_ref[...] = z * _dinv(d0_ref[...], d1_ref[...])

    return pl.pallas_call(
        body,
        grid=(_G,),
        in_specs=[
            pl.BlockSpec((_R, 256), lambda i: (i, 0)),
            pl.BlockSpec((2, 256), lambda i: (0, 0)),
            pl.BlockSpec((1, 256), lambda i: (0, 0)),
            pl.BlockSpec((1, 256), lambda i: (0, 0)),
            pl.BlockSpec((1, 1), lambda i: (0, 0)),
            pl.BlockSpec((256, 128), lambda i: (0, 0)),
            pl.BlockSpec((_R, _D), lambda i: (i, 0)),
            pl.BlockSpec((_R, _D), lambda i: (i, 0)),
        ],
        out_specs=pl.BlockSpec((_R, _D), lambda i: (i, 0)),
        out_shape=jax.ShapeDtypeStruct((_N, _D), jnp.float32),
    )(y, st, gammar, betar, par, W2, deg1a, deg1b)


def _t4_final(agg1a, agg1b, zp, deg1a, deg1b, b2r):
    def body(a0_ref, a1_ref, zp_ref, d0_ref, d1_ref, b_ref, o_ref):
        dinv = _dinv(d0_ref[...], d1_ref[...])
        o_ref[...] = (a0_ref[...] + a1_ref[...] + zp_ref[...]) * dinv + b_ref[...]

    return pl.pallas_call(
        body,
        grid=(_G,),
        in_specs=[
            pl.BlockSpec((_R, _D), lambda i: (i, 0)),
            pl.BlockSpec((_R, _D), lambda i: (i, 0)),
            pl.BlockSpec((_R, _D), lambda i: (i, 0)),
            pl.BlockSpec((_R, _D), lambda i: (i, 0)),
            pl.BlockSpec((_R, _D), lambda i: (i, 0)),
            pl.BlockSpec((1, _D), lambda i: (0, 0)),
        ],
        out_specs=pl.BlockSpec((_R, _D), lambda i: (i, 0)),
        out_shape=jax.ShapeDtypeStruct((_N, _D), jnp.float32),
    )(agg1a, agg1b, zp, deg1a, deg1b, b2r)


def kernel(x, edge_index_0, edge_index_1, size_0, size_1,
           W1, b1, W2, b2, bn_gamma, bn_beta, prelu_a):
    src0 = edge_index_0[0].astype(jnp.int32)
    dst0 = edge_index_0[1].astype(jnp.int32)
    src1 = edge_index_1[0].astype(jnp.int32)
    dst1 = edge_index_1[1].astype(jnp.int32)

    zrows = jnp.zeros((_ZCH, _D), jnp.float32)
    orows = jnp.ones((_CH, _D), jnp.float32)

    deg0, deg1 = _sc_degrees(dst0, dst1, zrows, orows)
    deg0a, deg0b = deg0[:_N], deg0[_NP:_NP + _N]
    deg1a, deg1b = deg1[:_N], deg1[_NP:_NP + _N]
    xp = _t1_scale(deg0a, deg0b, x)
    agg0 = _sc_aggregate(src0, dst0, xp, zrows)
    agg0a, agg0b = agg0[:_N], agg0[_NP:_NP + _N]
    y, st = _t2_layer1(agg0a, agg0b, xp, deg0a, deg0b, W1, b1.reshape(1, -1))
    zp = _t3_layer2_in(y, st, bn_gamma.reshape(1, -1), bn_beta.reshape(1, -1),
                       prelu_a.reshape(1, 1), W2, deg1a, deg1b)
    agg1 = _sc_aggregate(src1, dst1, zp, zrows)
    agg1a, agg1b = agg1[:_N], agg1[_NP:_NP + _N]
    out = _t4_final(agg1a, agg1b, zp, deg1a, deg1b, b2.reshape(1, -1))
    return out
